# baseline (speedup 1.0000x reference)
import functools

import jax
import jax.numpy as jnp
from jax import lax
from jax.experimental import pallas as pl
from jax.experimental.pallas import tpu as pltpu
from jax.experimental.pallas import tpu_sc as plsc

DIM = 128
EPS = 1e-07

_NC = 2
_NS = 16
_NW = _NC * _NS
_CHUNK = 128
_LANES = 16
_KG = DIM // _LANES

def _lane_gather(v, idx):
    dn = lax.GatherDimensionNumbers(
        offset_dims=(), collapsed_slice_dims=(0,), start_index_map=(0,))
    return lax.gather(v, idx[:, None], dn, slice_sizes=(1,),
                      mode=lax.GatherScatterMode.PROMISE_IN_BOUNDS)


def _iota16():
    return lax.iota(jnp.int32, _LANES)


def _butterfly_sum(v):
    for sh in (8, 4, 2, 1):
        v = v + _lane_gather(v, lax.bitwise_xor(_iota16(), jnp.int32(sh)))
    return v


def _splat(v, j):
    return _lane_gather(v, lax.bitwise_and(_iota16(), jnp.int32(0))
                        + jnp.int32(j))


def _rsqrt(v):
    i = lax.bitcast_convert_type(v, jnp.int32)
    i = jnp.int32(0x5F3759DF) - lax.shift_right_logical(i, 1)
    y = lax.bitcast_convert_type(i, jnp.float32)
    xh = v * jnp.float32(0.5)
    for _ in range(3):
        y = y * (jnp.float32(1.5) - xh * y * y)
    return y


def _sc_fused(ids_flat, ttf_flat, word_table, pos2, dt, gamma, beta, s_len):
    n_tok = ids_flat.shape[0]
    assert n_tok % _NW == 0
    per_w = n_tok // _NW
    assert per_w % _CHUNK == 0
    n_chunks = per_w // _CHUNK
    n_super = (n_chunks - 2) // 3
    assert n_chunks == 3 * n_super + 2
    assert (_CHUNK % 8 == 0) and (s_len % 8 == 0)

    mesh = plsc.VectorSubcoreMesh(core_axis_name="c", subcore_axis_name="s")

    @functools.partial(
        pl.kernel,
        out_type=jax.ShapeDtypeStruct((n_tok, DIM), jnp.float32),
        mesh=mesh,
        scratch_types=[
            pltpu.VMEM((per_w,), jnp.int32),
            pltpu.VMEM((per_w,), jnp.float32),
            pltpu.VMEM((2 * s_len, DIM), jnp.float32),
            pltpu.VMEM((DIM,), jnp.float32),
            pltpu.VMEM((DIM,), jnp.float32),
            pltpu.VMEM((DIM,), jnp.float32),
            pltpu.VMEM((_CHUNK, DIM), jnp.float32),
            pltpu.VMEM((_CHUNK, DIM), jnp.float32),
            pltpu.VMEM((_CHUNK, DIM), jnp.float32),
            pltpu.SemaphoreType.DMA,
            pltpu.SemaphoreType.DMA,
            pltpu.SemaphoreType.DMA,
            pltpu.SemaphoreType.DMA,
            pltpu.SemaphoreType.DMA,
            pltpu.SemaphoreType.DMA,
        ],
    )
    def fused(ids_hbm, ttf_hbm, table_hbm, pos_hbm, dt_hbm, g_hbm, b_hbm,
              out_hbm, ids_v, ttf_v, pos_v, dt_v, g_v, b_v,
              buf0, buf1, buf2, gs0, gs1, gs2, os0, os1, os2):
        wid = lax.axis_index("s") * _NC + lax.axis_index("c")
        base = wid * per_w

        pltpu.sync_copy(ids_hbm.at[pl.ds(base, per_w)], ids_v)
        pltpu.sync_copy(ttf_hbm.at[pl.ds(base, per_w)], ttf_v)
        pltpu.sync_copy(pos_hbm, pos_v)
        pltpu.sync_copy(dt_hbm, dt_v)
        pltpu.sync_copy(g_hbm, g_v)
        pltpu.sync_copy(b_hbm, b_v)

        dts = [dt_v[pl.ds(k * _LANES, _LANES)] for k in range(_KG)]
        gms = [g_v[pl.ds(k * _LANES, _LANES)] for k in range(_KG)]
        bts = [b_v[pl.ds(k * _LANES, _LANES)] for k in range(_KG)]

        bufs = (buf0, buf1, buf2)
        gsem = (gs0, gs1, gs2)
        osem = (os0, os1, os2)

        def start_gather(c, bi):
            pltpu.async_copy(
                table_hbm.at[ids_v.at[pl.ds(c * _CHUNK, _CHUNK)]],
                bufs[bi], gsem[bi])

        def wait_gather(bi):
            pltpu.make_async_copy(
                out_hbm.at[pl.ds(0, _CHUNK)], bufs[bi], gsem[bi]).wait()

        def start_out(c, bi):
            pltpu.async_copy(
                bufs[bi], out_hbm.at[pl.ds(base + c * _CHUNK, _CHUNK)],
                osem[bi])

        def wait_out(bi):
            pltpu.make_async_copy(
                bufs[bi], out_hbm.at[pl.ds(0, _CHUNK)], osem[bi]).wait()

        inv_dim = jnp.float32(1.0 / DIM)

        def compute_chunk(c, bi):
            buf = bufs[bi]
            pbase = lax.rem(c * _CHUNK, s_len)
            cbase = c * _CHUNK

            def group(gi, carry):
                tb = gi * _LANES
                ttv = ttf_v[pl.ds(cbase + tb, _LANES)]
                for j in range(_LANES):
                    t = tb + j
                    prow = pbase + t
                    x = [buf[t, pl.ds(k * _LANES, _LANES)]
                         + pos_v[prow, pl.ds(k * _LANES, _LANES)]
                         for k in range(_KG)]
                    tsp = _splat(ttv, j)
                    x = [x[k] + tsp * dts[k] for k in range(_KG)]
                    s = (((x[0] + x[1]) + (x[2] + x[3]))
                         + ((x[4] + x[5]) + (x[6] + x[7])))
                    s = _butterfly_sum(s)
                    xx = [x[k] * x[k] for k in range(_KG)]
                    q = (((xx[0] + xx[1]) + (xx[2] + xx[3]))
                         + ((xx[4] + xx[5]) + (xx[6] + xx[7])))
                    q = _butterfly_sum(q)
                    mu = s * inv_dim
                    var = q * inv_dim - mu * mu
                    rin = _rsqrt(var + jnp.float32(EPS))
                    for k in range(_KG):
                        buf[t, pl.ds(k * _LANES, _LANES)] = (
                            (x[k] - mu) * rin * gms[k] + bts[k])
                return carry

            lax.fori_loop(0, _CHUNK // _LANES, group, 0)

        start_gather(0, 0)
        start_gather(1, 1)

        def super_body(sp, carry):
            c0 = sp * 3
            wait_gather(0)
            compute_chunk(c0, 0)
            start_out(c0, 0)

            @pl.when(sp > 0)
            def _():
                wait_out(2)
            start_gather(c0 + 2, 2)

            wait_gather(1)
            compute_chunk(c0 + 1, 1)
            start_out(c0 + 1, 1)
            wait_out(0)
            start_gather(c0 + 3, 0)

            wait_gather(2)
            compute_chunk(c0 + 2, 2)
            start_out(c0 + 2, 2)
            wait_out(1)
            start_gather(c0 + 4, 1)
            return carry

        lax.fori_loop(0, n_super, super_body, 0)

        ce = n_super * 3
        wait_gather(0)
        compute_chunk(ce, 0)
        start_out(ce, 0)
        wait_gather(1)
        compute_chunk(ce + 1, 1)
        start_out(ce + 1, 1)
        wait_out(2)
        wait_out(0)
        wait_out(1)

    return fused(ids_flat, ttf_flat, word_table, pos2, dt, gamma, beta)


def _tc_pos_body(pos_ref, out_ref):
    out_ref[...] = jnp.broadcast_to(pos_ref[...][None], out_ref.shape)


def kernel(input_ids, token_type_ids, word_table, pos_table, type_table, gamma, beta):
    b, s = input_ids.shape
    ids_flat = input_ids.reshape(-1).astype(jnp.int32)
    ttf_flat = token_type_ids.reshape(-1).astype(jnp.float32)

    pos_s = pos_table[:s]
    slab = pos_s + type_table[0][None, :]
    pos2 = jnp.concatenate([slab, slab], axis=0)
    dt = type_table[1] - type_table[0]

    emb = _sc_fused(ids_flat, ttf_flat, word_table, pos2, dt, gamma, beta,
                    s).reshape(b, s, DIM)

    br = 8
    pos_out = pl.pallas_call(
        _tc_pos_body,
        grid=(b // br,),
        in_specs=[pl.BlockSpec((s, DIM), lambda i: (0, 0))],
        out_specs=pl.BlockSpec((br, s, DIM), lambda i: (i, 0, 0)),
        out_shape=jax.ShapeDtypeStruct((b, s, DIM), jnp.float32),
    )(pos_s)

    return emb, pos_out


# speedup vs baseline: 1.5149x; 1.5149x over previous
import functools

import jax
import jax.numpy as jnp
from jax import lax
from jax.experimental import pallas as pl
from jax.experimental.pallas import tpu as pltpu
from jax.experimental.pallas import tpu_sc as plsc

DIM = 128
EPS = 1e-07

_NC = 2
_NS = 16
_NW = _NC * _NS
_CHUNK = 128


def _sc_gather(ids_flat, word_table):
    n_tok = ids_flat.shape[0]
    assert n_tok % (_NW * _CHUNK) == 0
    per_w = n_tok // _NW
    n_chunks = per_w // _CHUNK
    n_super = (n_chunks - 2) // 3
    assert n_chunks == 3 * n_super + 2

    mesh = plsc.VectorSubcoreMesh(core_axis_name="c", subcore_axis_name="s")

    @functools.partial(
        pl.kernel,
        out_type=jax.ShapeDtypeStruct((n_tok, DIM), jnp.float32),
        mesh=mesh,
        scratch_types=[
            pltpu.VMEM((per_w,), jnp.int32),
            pltpu.VMEM((_CHUNK, DIM), jnp.float32),
            pltpu.VMEM((_CHUNK, DIM), jnp.float32),
            pltpu.VMEM((_CHUNK, DIM), jnp.float32),
            pltpu.SemaphoreType.DMA,
            pltpu.SemaphoreType.DMA,
            pltpu.SemaphoreType.DMA,
            pltpu.SemaphoreType.DMA,
            pltpu.SemaphoreType.DMA,
            pltpu.SemaphoreType.DMA,
        ],
    )
    def gather_kernel(ids_hbm, table_hbm, out_hbm, ids_v,
                      buf0, buf1, buf2, gs0, gs1, gs2, os0, os1, os2):
        wid = lax.axis_index("s") * _NC + lax.axis_index("c")
        base = wid * per_w

        pltpu.sync_copy(ids_hbm.at[pl.ds(base, per_w)], ids_v)

        bufs = (buf0, buf1, buf2)
        gsem = (gs0, gs1, gs2)
        osem = (os0, os1, os2)

        def start_gather(c, bi):
            pltpu.async_copy(
                table_hbm.at[ids_v.at[pl.ds(c * _CHUNK, _CHUNK)]],
                bufs[bi], gsem[bi])

        def wait_gather(bi):
            pltpu.make_async_copy(
                out_hbm.at[pl.ds(0, _CHUNK)], bufs[bi], gsem[bi]).wait()

        def start_out(c, bi):
            pltpu.async_copy(
                bufs[bi], out_hbm.at[pl.ds(base + c * _CHUNK, _CHUNK)],
                osem[bi])

        def wait_out(bi):
            pltpu.make_async_copy(
                bufs[bi], out_hbm.at[pl.ds(0, _CHUNK)], osem[bi]).wait()

        start_gather(0, 0)
        start_gather(1, 1)

        def super_body(sp, carry):
            c0 = sp * 3
            wait_gather(0)
            start_out(c0, 0)

            @pl.when(sp > 0)
            def _():
                wait_out(2)
            start_gather(c0 + 2, 2)

            wait_gather(1)
            start_out(c0 + 1, 1)
            wait_out(0)
            start_gather(c0 + 3, 0)

            wait_gather(2)
            start_out(c0 + 2, 2)
            wait_out(1)
            start_gather(c0 + 4, 1)
            return carry

        lax.fori_loop(0, n_super, super_body, 0)

        ce = n_super * 3
        wait_gather(0)
        start_out(ce, 0)
        wait_gather(1)
        start_out(ce + 1, 1)
        wait_out(2)
        wait_out(0)
        wait_out(1)

    return gather_kernel(ids_flat, word_table)


def _tc_body(g_ref, tt_ref, pos_ref, ty_ref, gb_ref, emb_ref):
    x = g_ref[...]
    pos = pos_ref[...]
    ty = ty_ref[...]
    gb = gb_ref[...]
    ttf = tt_ref[...]

    x = x + pos[None]
    t0 = ty[0][None, None, :]
    dt = (ty[1] - ty[0])[None, None, :]
    x = x + t0 + ttf[:, :, None] * dt

    mu = jnp.mean(x, axis=-1, keepdims=True)
    xc = x - mu
    var = jnp.mean(xc * xc, axis=-1, keepdims=True)
    y = xc * lax.rsqrt(var + EPS)
    y = y * gb[0][None, None, :] + gb[1][None, None, :]

    emb_ref[...] = y


def _tc_pos_body(pos_ref, out_ref):
    out_ref[...] = jnp.broadcast_to(pos_ref[...][None], out_ref.shape)


def kernel(input_ids, token_type_ids, word_table, pos_table, type_table, gamma, beta):
    b, s = input_ids.shape
    ids_flat = input_ids.reshape(-1).astype(jnp.int32)
    gathered = _sc_gather(ids_flat, word_table).reshape(b, s, DIM)

    ttf = token_type_ids.astype(jnp.float32)
    pos_s = pos_table[:s]
    gb = jnp.stack([gamma, beta])

    br = 8
    grid = (b // br,)
    emb = pl.pallas_call(
        _tc_body,
        grid=grid,
        in_specs=[
            pl.BlockSpec((br, s, DIM), lambda i: (i, 0, 0)),
            pl.BlockSpec((br, s), lambda i: (i, 0)),
            pl.BlockSpec((s, DIM), lambda i: (0, 0)),
            pl.BlockSpec((2, DIM), lambda i: (0, 0)),
            pl.BlockSpec((2, DIM), lambda i: (0, 0)),
        ],
        out_specs=pl.BlockSpec((br, s, DIM), lambda i: (i, 0, 0)),
        out_shape=jax.ShapeDtypeStruct((b, s, DIM), jnp.float32),
    )(gathered, ttf, pos_s, type_table, gb)

    pos_out = pl.pallas_call(
        _tc_pos_body,
        grid=grid,
        in_specs=[pl.BlockSpec((s, DIM), lambda i: (0, 0))],
        out_specs=pl.BlockSpec((br, s, DIM), lambda i: (i, 0, 0)),
        out_shape=jax.ShapeDtypeStruct((b, s, DIM), jnp.float32),
    )(pos_s)

    return emb, pos_out
